# baseline (speedup 1.0000x reference)
import functools

import jax
import jax.numpy as jnp
from jax import lax
from jax.experimental import pallas as pl
from jax.experimental.pallas import tpu as pltpu
from jax.experimental.pallas import tpu_sc as plsc

B, F, D, V = 4096, 26, 64, 100000
NC, NS = 2, 16
NW = NC * NS
BPW = B // NW
BB = 256


def _sc_gather(item_idx, table):
    mesh = plsc.VectorSubcoreMesh(
        core_axis_name="c", subcore_axis_name="s", num_cores=NC, num_subcores=NS
    )

    @functools.partial(
        pl.kernel,
        out_type=jax.ShapeDtypeStruct((B, F), jnp.float32),
        mesh=mesh,
        scratch_types=[
            pltpu.VMEM((BPW,), jnp.int32),
            pltpu.VMEM((BPW, F), jnp.float32),
            pltpu.SemaphoreType.DMA,
        ],
        compiler_params=pltpu.CompilerParams(use_tc_tiling_on_sc=False),
    )
    def gather_kernel(idx_hbm, table_hbm, out_hbm, idx_v, rows_v, sem):
        wid = lax.axis_index("s") * NC + lax.axis_index("c")
        base = wid * BPW
        pltpu.sync_copy(idx_hbm.at[pl.ds(base, BPW)], idx_v)
        pltpu.async_copy(table_hbm.at[idx_v], rows_v, sem).wait()
        pltpu.sync_copy(rows_v, out_hbm.at[pl.ds(base, BPW)])

    return gather_kernel(item_idx, table)


def _pool_body(alpha_ref, stack_ref, out_ref):
    a = alpha_ref[...]
    m = jnp.max(a, axis=1, keepdims=True)
    e = jnp.exp(a - m)
    w = e / jnp.sum(e, axis=1, keepdims=True)
    x = stack_ref[...]
    out_ref[...] = jnp.sum(w[:, :, None] * x, axis=1)


def _tc_pool(alpha_rows, stack_embedding):
    grid = (B // BB,)
    return pl.pallas_call(
        _pool_body,
        grid=grid,
        in_specs=[
            pl.BlockSpec((BB, F), lambda i: (i, 0)),
            pl.BlockSpec((BB, F, D), lambda i: (i, 0, 0)),
        ],
        out_specs=pl.BlockSpec((BB, D), lambda i: (i, 0)),
        out_shape=jax.ShapeDtypeStruct((B, D), jnp.float32),
    )(alpha_rows, stack_embedding)


def kernel(stack_embedding, item_input, alpha_embeddings):
    item_idx = jnp.reshape(item_input, (B,)).astype(jnp.int32)
    alpha_rows = _sc_gather(item_idx, alpha_embeddings)
    return _tc_pool(alpha_rows, stack_embedding)


# speedup vs baseline: 1.4834x; 1.4834x over previous
import functools

import jax
import jax.numpy as jnp
from jax import lax
from jax.experimental import pallas as pl
from jax.experimental.pallas import tpu as pltpu
from jax.experimental.pallas import tpu_sc as plsc

B, F, D, V = 4096, 26, 64, 100000
NC, NS = 2, 16
NW = NC * NS
BPW = B // NW
BB = 256


def _sc_gather(item_idx, table):
    mesh = plsc.VectorSubcoreMesh(
        core_axis_name="c", subcore_axis_name="s", num_cores=NC, num_subcores=NS
    )

    @functools.partial(
        pl.kernel,
        out_type=jax.ShapeDtypeStruct((B, F), jnp.float32),
        mesh=mesh,
        scratch_types=[
            pltpu.VMEM((BPW,), jnp.int32),
            pltpu.VMEM((BPW, F), jnp.float32),
            pltpu.SemaphoreType.DMA,
        ],
        compiler_params=pltpu.CompilerParams(use_tc_tiling_on_sc=False),
    )
    def gather_kernel(idx_hbm, table_hbm, out_hbm, idx_v, rows_v, sem):
        wid = lax.axis_index("s") * NC + lax.axis_index("c")
        base = wid * BPW
        pltpu.sync_copy(idx_hbm.at[pl.ds(base, BPW)], idx_v)
        pltpu.async_copy(table_hbm.at[idx_v], rows_v, sem).wait()
        pltpu.sync_copy(rows_v, out_hbm.at[pl.ds(base, BPW)])

    return gather_kernel(item_idx, table)


def _pool_body(alpha_ref, stack_ref, out_ref):
    a = alpha_ref[...]
    m = jnp.max(a, axis=1, keepdims=True)
    e = jnp.exp(a - m)
    w = e / jnp.sum(e, axis=1, keepdims=True)
    x = stack_ref[...]
    out_ref[...] = jnp.sum(w[:, :, None] * x, axis=1)


def _tc_pool(alpha_rows, stack_embedding):
    grid = (B // BB,)
    return pl.pallas_call(
        _pool_body,
        grid=grid,
        in_specs=[
            pl.BlockSpec((BB, F), lambda i: (i, 0)),
            pl.BlockSpec((BB, F, D), lambda i: (i, 0, 0)),
        ],
        out_specs=pl.BlockSpec((BB, D), lambda i: (i, 0)),
        out_shape=jax.ShapeDtypeStruct((B, D), jnp.float32),
    )(alpha_rows, stack_embedding)


def kernel(stack_embedding, item_input, alpha_embeddings):
    item_idx = jnp.reshape(item_input, (B,)).astype(jnp.int32)
    alpha_rows = jnp.take(alpha_embeddings, item_idx, axis=0)
    return _tc_pool(alpha_rows, stack_embedding)
